# baseline (speedup 1.0000x reference)
import functools

import jax
import jax.numpy as jnp
from jax import lax
from jax.experimental import pallas as pl
from jax.experimental.pallas import tpu as pltpu
from jax.experimental.pallas import tpu_sc as plsc

NC = 2
NS = 16
NW = NC * NS
NPW = 320
DUMMY = NPW
C0 = 4000
CS = 1024
GB = 16
ECAP = 2048 * 80

NEG_INF = float("-inf")


def _worker_id():
    return lax.axis_index("s") * NC + lax.axis_index("c")


def _bucket_body(src_hbm, dst_hbm, ew_hbm, bsrc_hbm, bdst_hbm, bew_hbm,
                 cnt_hbm, srcc, dstc, ewc, csrc, cdst, cew, cntv):
    w = _worker_id()
    lo = w * NPW
    lanes = lax.iota(jnp.int32, 16)
    nchunks = src_hbm.shape[0] // C0

    def chunk_body(chunk, total):
        base = pl.multiple_of(chunk * C0, 8)
        pltpu.sync_copy(src_hbm.at[pl.ds(base, C0)], srcc)
        pltpu.sync_copy(dst_hbm.at[pl.ds(base, C0)], dstc)
        pltpu.sync_copy(ew_hbm.at[pl.ds(base, C0)], ewc)

        def scan_body(i, m):
            off = i * 16
            dv = dstc[pl.ds(off, 16)]
            msk = (dv >= lo) & (dv < lo + NPW)
            nmatch = plsc.all_reduce_population_count(msk)[0]

            def compact():
                cnt16 = jnp.where(msk, 1, 0)
                pos = m + plsc.cumsum(cnt16) - 1
                plsc.store_scatter(cdst, [pos], dv - lo, mask=msk)
                sv = srcc[pl.ds(off, 16)]
                plsc.store_scatter(csrc, [pos], sv, mask=msk)
                ev = ewc[pl.ds(off, 16)]
                plsc.store_scatter(cew, [pos], ev, mask=msk)

            pl.when(nmatch > 0)(compact)
            return m + nmatch

        m = lax.fori_loop(0, C0 // 16, scan_body, jnp.int32(0))
        cdst[pl.ds(m, 16)] = jnp.full((16,), DUMMY, jnp.int32)
        cdst[pl.ds(m + 16, 16)] = jnp.full((16,), DUMMY, jnp.int32)
        csrc[pl.ds(m, 16)] = jnp.zeros((16,), jnp.int32)
        csrc[pl.ds(m + 16, 16)] = jnp.zeros((16,), jnp.int32)
        cew[pl.ds(m, 16)] = jnp.zeros((16,), jnp.float32)
        cew[pl.ds(m + 16, 16)] = jnp.zeros((16,), jnp.float32)
        m_p = ((m + GB - 1) // GB) * GB

        def flush_body(k, _):
            o = k * 16
            fo = pl.multiple_of(w * ECAP + total + o, 8)
            pltpu.sync_copy(csrc.at[pl.ds(o, 16)], bsrc_hbm.at[pl.ds(fo, 16)])
            pltpu.sync_copy(cdst.at[pl.ds(o, 16)], bdst_hbm.at[pl.ds(fo, 16)])
            pltpu.sync_copy(cew.at[pl.ds(o, 16)], bew_hbm.at[pl.ds(fo, 16)])
            return 0

        lax.fori_loop(0, m_p // 16, flush_body, 0)
        return total + m_p

    total = lax.fori_loop(0, nchunks, chunk_body, jnp.int32(0))
    cntv[...] = jnp.where(lanes == 0, total, 0)
    pltpu.sync_copy(cntv, cnt_hbm.at[pl.ds(pl.multiple_of(w * 16, 8), 16)])


def _make_bucket(E):
    mesh = plsc.VectorSubcoreMesh(core_axis_name="c", subcore_axis_name="s")
    return pl.kernel(
        _bucket_body,
        out_type=(
            jax.ShapeDtypeStruct((NW * ECAP,), jnp.int32),
            jax.ShapeDtypeStruct((NW * ECAP,), jnp.int32),
            jax.ShapeDtypeStruct((NW * ECAP,), jnp.float32),
            jax.ShapeDtypeStruct((NW * 16,), jnp.int32),
        ),
        mesh=mesh,
        compiler_params=pltpu.CompilerParams(needs_layout_passes=False),
        scratch_types=[
            pltpu.VMEM((C0,), jnp.int32),
            pltpu.VMEM((C0,), jnp.int32),
            pltpu.VMEM((C0,), jnp.float32),
            pltpu.VMEM((C0 + GB,), jnp.int32),
            pltpu.VMEM((C0 + GB,), jnp.int32),
            pltpu.VMEM((C0 + GB,), jnp.float32),
            pltpu.VMEM((16,), jnp.int32),
        ],
    )


def _agg_body(D, feat_hbm, bsrc_hbm, bdst_hbm, bew_hbm, cnt_hbm, *rest):
    nout = 4 if D > 128 else 2
    agg_hbms = rest[:nout]
    (shared, agg0, agg1, rows_a, rows_b,
     srcs, dsts, ews, cntv, sem_a, sem_b) = rest[nout:]
    w = _worker_id()
    lo = w * NPW
    s_id = lax.axis_index("s")
    DP = min(D, 128)
    D2 = DP // 2
    npass = max(1, D // 128)
    aggs = (agg0, agg1)

    pltpu.sync_copy(cnt_hbm.at[pl.ds(pl.multiple_of(w * 16, 8), 16)], cntv)
    total = cntv[...][0]

    def fire(g, rows, sem):
        sv = srcs[pl.ds(g * GB, GB)]
        return pltpu.async_copy(shared.at[sv], rows, sem)

    def process(g, rows):
        off = g * GB
        for q in range(GB // 16):
            av = dsts[pl.ds(off + q * 16, 16)] * D2
            ev = ews[pl.ds(off + q * 16, 16)]
            for e in range(16):
                e_s = ev[e]
                abase = av[e]
                for k in range(2):
                    for j in range(D2 // 16):
                        a = aggs[k][pl.ds(abase + j * 16, 16)]
                        r = rows[q * 16 + e, pl.ds(k * D2 + j * 16, 16)]
                        aggs[k][pl.ds(abase + j * 16, 16)] = jnp.maximum(
                            a, r * e_s)

    def chunk_body(c, _):
        cbase = c * CS
        co = pl.multiple_of(w * ECAP + cbase, 8)
        pltpu.sync_copy(bsrc_hbm.at[pl.ds(co, CS)], srcs)
        pltpu.sync_copy(bdst_hbm.at[pl.ds(co, CS)], dsts)
        pltpu.sync_copy(bew_hbm.at[pl.ds(co, CS)], ews)
        nb = jnp.minimum(CS, total - cbase) // GB

        @pl.when(nb > 0)
        def _():
            fire(0, rows_a, sem_a)

        def pair_body(p, _):
            ga = 2 * p
            gb = 2 * p + 1

            @pl.when(gb < nb)
            def _():
                fire(gb, rows_b, sem_b)

            pltpu.make_async_copy(shared.at[srcs[pl.ds(0, GB)]],
                                  rows_a, sem_a).wait()
            process(ga, rows_a)

            @pl.when(ga + 2 < nb)
            def _():
                fire(ga + 2, rows_a, sem_a)

            @pl.when(gb < nb)
            def _():
                pltpu.make_async_copy(shared.at[srcs[pl.ds(0, GB)]],
                                      rows_b, sem_b).wait()
                process(gb, rows_b)

            return 0

        lax.fori_loop(0, (nb + 1) // 2, pair_body, 0)
        return 0

    ro = pl.multiple_of(s_id * (NW * NPW // NS), 8)
    nrs = NW * NPW // NS
    for p in range(npass):
        if p > 0:
            plsc.subcore_barrier()
        if D > 128:
            pltpu.sync_copy(feat_hbm.at[pl.ds(ro, nrs), pl.ds(128 * p, 128)],
                            shared.at[pl.ds(ro, nrs)])
        else:
            pltpu.sync_copy(feat_hbm.at[pl.ds(ro, nrs)],
                            shared.at[pl.ds(ro, nrs)])
        plsc.subcore_barrier()

        def init_body(i, _):
            ninf = jnp.full((16,), NEG_INF, jnp.float32)
            for k in range(2):
                aggs[k][pl.ds(i * 16, 16)] = ninf
            return 0

        lax.fori_loop(0, (NPW + 1) * D2 // 16, init_body, 0)

        nchunks = (total + CS - 1) // CS
        lax.fori_loop(0, nchunks, chunk_body, 0)

        for k in range(2):
            pltpu.sync_copy(
                aggs[k].at[pl.ds(0, NPW * D2)],
                agg_hbms[2 * p + k].at[
                    pl.ds(pl.multiple_of(lo * D2, 8), NPW * D2)])


def _make_agg(D):
    nout = 4 if D > 128 else 2
    D2 = min(D, 128) // 2
    mesh = plsc.VectorSubcoreMesh(core_axis_name="c", subcore_axis_name="s")
    return pl.kernel(
        functools.partial(_agg_body, D),
        out_type=tuple(
            jax.ShapeDtypeStruct((NW * NPW * D2,), jnp.float32)
            for _ in range(nout)),
        mesh=mesh,
        compiler_params=pltpu.CompilerParams(needs_layout_passes=False),
        scratch_types=[
            pltpu.VMEM_SHARED((NW * NPW, 128), jnp.float32),
            pltpu.VMEM(((NPW + 1) * D2,), jnp.float32),
            pltpu.VMEM(((NPW + 1) * D2,), jnp.float32),
            pltpu.VMEM((GB, 128), jnp.float32),
            pltpu.VMEM((GB, 128), jnp.float32),
            pltpu.VMEM((CS,), jnp.int32),
            pltpu.VMEM((CS,), jnp.int32),
            pltpu.VMEM((CS,), jnp.float32),
            pltpu.VMEM((16,), jnp.int32),
            pltpu.SemaphoreType.DMA,
            pltpu.SemaphoreType.DMA,
        ],
    )


def _make_linear_body(nagg):
    def body(*refs):
        x_ref = refs[0]
        a_refs = refs[1:1 + nagg]
        w_ref, b_ref, o_ref = refs[1 + nagg:]
        a = jnp.concatenate([r[...] for r in a_refs], axis=1)
        a = jnp.where(jnp.isneginf(a), 0.0, a)
        x = x_ref[...] + a
        y = jnp.dot(x, w_ref[...], preferred_element_type=jnp.float32)
        o_ref[...] = jnp.maximum(y + b_ref[...], 0.0)
    return body


def _linear(x, aggs, W, b):
    n, din = x.shape
    dk = din // len(aggs)
    dout = W.shape[1]
    bn = 1000
    aspec = pl.BlockSpec((bn, dk), lambda i: (i, 0))
    return pl.pallas_call(
        _make_linear_body(len(aggs)),
        grid=(n // bn,),
        in_specs=[pl.BlockSpec((bn, din), lambda i: (i, 0))]
        + [aspec] * len(aggs)
        + [
            pl.BlockSpec((din, dout), lambda i: (0, 0)),
            pl.BlockSpec((1, dout), lambda i: (0, 0)),
        ],
        out_specs=pl.BlockSpec((bn, dout), lambda i: (i, 0)),
        out_shape=jax.ShapeDtypeStruct((n, dout), jnp.float32),
    )(x, *aggs, W, b.reshape(1, dout))


def kernel(feat, edge_index, edge_weight, W1, b1, W2, b2, W3, b3, W4, b4):
    n = feat.shape[0]
    src = edge_index[0]
    dst = edge_index[1]
    bsrc, bdst, bew, cnt = _make_bucket(src.shape[0])(src, dst, edge_weight)
    h = feat
    for W, b in ((W1, b1), (W2, b2), (W3, b3), (W4, b4)):
        din = W.shape[0]
        hg = jnp.pad(h, ((0, NW * NPW - n), (0, 128 - din if din < 128 else 0)))
        aggs = _make_agg(din)(hg, bsrc, bdst, bew, cnt)
        dk = min(din, 128) // 2
        aggs = [a.reshape(NW * NPW, dk)[:n] for a in aggs]
        h = _linear(h, aggs, W, b)
    return h


# speedup vs baseline: 1.0626x; 1.0626x over previous
import functools

import jax
import jax.numpy as jnp
from jax import lax
from jax.experimental import pallas as pl
from jax.experimental.pallas import tpu as pltpu
from jax.experimental.pallas import tpu_sc as plsc

NC = 2
NS = 16
NW = NC * NS
NPW = 320
DUMMY = NPW
C0 = 4000
CS = 1024
GB = 16
ECAP = 2048 * 80

NEG_INF = float("-inf")


def _worker_id():
    return lax.axis_index("s") * NC + lax.axis_index("c")


def _bucket_body(src_hbm, dst_hbm, ew_hbm, bsrc_hbm, bdst_hbm, bew_hbm,
                 cnt_hbm, srcc, dstc, ewc, csrc, cdst, cew, cntv):
    w = _worker_id()
    lo = w * NPW
    lanes = lax.iota(jnp.int32, 16)
    nchunks = src_hbm.shape[0] // C0

    def chunk_body(chunk, total):
        base = pl.multiple_of(chunk * C0, 8)
        pltpu.sync_copy(src_hbm.at[pl.ds(base, C0)], srcc)
        pltpu.sync_copy(dst_hbm.at[pl.ds(base, C0)], dstc)
        pltpu.sync_copy(ew_hbm.at[pl.ds(base, C0)], ewc)

        def scan_body(i, m):
            off = i * 16
            dv = dstc[pl.ds(off, 16)]
            msk = (dv >= lo) & (dv < lo + NPW)
            cnt16 = jnp.where(msk, 1, 0)
            cum = plsc.cumsum(cnt16)
            pos = m + cum - 1
            plsc.store_scatter(cdst, [pos], dv - lo, mask=msk)
            sv = srcc[pl.ds(off, 16)]
            plsc.store_scatter(csrc, [pos], sv, mask=msk)
            ev = ewc[pl.ds(off, 16)]
            plsc.store_scatter(cew, [pos], ev, mask=msk)
            return m + cum[15]

        m = lax.fori_loop(0, C0 // 16, scan_body, jnp.int32(0))
        cdst[pl.ds(m, 16)] = jnp.full((16,), DUMMY, jnp.int32)
        cdst[pl.ds(m + 16, 16)] = jnp.full((16,), DUMMY, jnp.int32)
        csrc[pl.ds(m, 16)] = jnp.zeros((16,), jnp.int32)
        csrc[pl.ds(m + 16, 16)] = jnp.zeros((16,), jnp.int32)
        cew[pl.ds(m, 16)] = jnp.zeros((16,), jnp.float32)
        cew[pl.ds(m + 16, 16)] = jnp.zeros((16,), jnp.float32)
        m_p = ((m + GB - 1) // GB) * GB

        def flush_body(k, _):
            o = k * 16
            fo = pl.multiple_of(w * ECAP + total + o, 8)
            pltpu.sync_copy(csrc.at[pl.ds(o, 16)], bsrc_hbm.at[pl.ds(fo, 16)])
            pltpu.sync_copy(cdst.at[pl.ds(o, 16)], bdst_hbm.at[pl.ds(fo, 16)])
            pltpu.sync_copy(cew.at[pl.ds(o, 16)], bew_hbm.at[pl.ds(fo, 16)])
            return 0

        lax.fori_loop(0, m_p // 16, flush_body, 0)
        return total + m_p

    total = lax.fori_loop(0, nchunks, chunk_body, jnp.int32(0))
    cntv[...] = jnp.where(lanes == 0, total, 0)
    pltpu.sync_copy(cntv, cnt_hbm.at[pl.ds(pl.multiple_of(w * 16, 8), 16)])


def _make_bucket(E):
    mesh = plsc.VectorSubcoreMesh(core_axis_name="c", subcore_axis_name="s")
    return pl.kernel(
        _bucket_body,
        out_type=(
            jax.ShapeDtypeStruct((NW * ECAP,), jnp.int32),
            jax.ShapeDtypeStruct((NW * ECAP,), jnp.int32),
            jax.ShapeDtypeStruct((NW * ECAP,), jnp.float32),
            jax.ShapeDtypeStruct((NW * 16,), jnp.int32),
        ),
        mesh=mesh,
        compiler_params=pltpu.CompilerParams(needs_layout_passes=False),
        scratch_types=[
            pltpu.VMEM((C0,), jnp.int32),
            pltpu.VMEM((C0,), jnp.int32),
            pltpu.VMEM((C0,), jnp.float32),
            pltpu.VMEM((C0 + GB,), jnp.int32),
            pltpu.VMEM((C0 + GB,), jnp.int32),
            pltpu.VMEM((C0 + GB,), jnp.float32),
            pltpu.VMEM((16,), jnp.int32),
        ],
    )


def _agg_body(D, feat_hbm, bsrc_hbm, bdst_hbm, bew_hbm, cnt_hbm, *rest):
    nout = 4 if D > 128 else 2
    agg_hbms = rest[:nout]
    (shared, agg0, agg1, rows_a, rows_b,
     srcs, dsts, ews, cntv, sem_a, sem_b) = rest[nout:]
    w = _worker_id()
    lo = w * NPW
    s_id = lax.axis_index("s")
    DP = min(D, 128)
    D2 = DP // 2
    npass = max(1, D // 128)
    aggs = (agg0, agg1)

    pltpu.sync_copy(cnt_hbm.at[pl.ds(pl.multiple_of(w * 16, 8), 16)], cntv)
    total = cntv[...][0]

    def fire(g, rows, sem):
        sv = srcs[pl.ds(g * GB, GB)]
        return pltpu.async_copy(shared.at[sv], rows, sem)

    def process(g, rows):
        off = g * GB
        for q in range(GB // 16):
            av = dsts[pl.ds(off + q * 16, 16)] * D2
            ev = ews[pl.ds(off + q * 16, 16)]
            for e in range(16):
                e_s = ev[e]
                abase = av[e]
                for k in range(2):
                    for j in range(D2 // 16):
                        a = aggs[k][pl.ds(abase + j * 16, 16)]
                        r = rows[q * 16 + e, pl.ds(k * D2 + j * 16, 16)]
                        aggs[k][pl.ds(abase + j * 16, 16)] = jnp.maximum(
                            a, r * e_s)

    def chunk_body(c, _):
        cbase = c * CS
        co = pl.multiple_of(w * ECAP + cbase, 8)
        pltpu.sync_copy(bsrc_hbm.at[pl.ds(co, CS)], srcs)
        pltpu.sync_copy(bdst_hbm.at[pl.ds(co, CS)], dsts)
        pltpu.sync_copy(bew_hbm.at[pl.ds(co, CS)], ews)
        nb = jnp.minimum(CS, total - cbase) // GB

        @pl.when(nb > 0)
        def _():
            fire(0, rows_a, sem_a)

        def pair_body(p, _):
            ga = 2 * p
            gb = 2 * p + 1

            @pl.when(gb < nb)
            def _():
                fire(gb, rows_b, sem_b)

            pltpu.make_async_copy(shared.at[srcs[pl.ds(0, GB)]],
                                  rows_a, sem_a).wait()
            process(ga, rows_a)

            @pl.when(ga + 2 < nb)
            def _():
                fire(ga + 2, rows_a, sem_a)

            @pl.when(gb < nb)
            def _():
                pltpu.make_async_copy(shared.at[srcs[pl.ds(0, GB)]],
                                      rows_b, sem_b).wait()
                process(gb, rows_b)

            return 0

        lax.fori_loop(0, (nb + 1) // 2, pair_body, 0)
        return 0

    ro = pl.multiple_of(s_id * (NW * NPW // NS), 8)
    nrs = NW * NPW // NS
    for p in range(npass):
        if p > 0:
            plsc.subcore_barrier()
        if D > 128:
            pltpu.sync_copy(feat_hbm.at[pl.ds(ro, nrs), pl.ds(128 * p, 128)],
                            shared.at[pl.ds(ro, nrs)])
        else:
            pltpu.sync_copy(feat_hbm.at[pl.ds(ro, nrs)],
                            shared.at[pl.ds(ro, nrs)])
        plsc.subcore_barrier()

        def init_body(i, _):
            ninf = jnp.full((16,), NEG_INF, jnp.float32)
            for k in range(2):
                aggs[k][pl.ds(i * 16, 16)] = ninf
            return 0

        lax.fori_loop(0, (NPW + 1) * D2 // 16, init_body, 0)

        nchunks = (total + CS - 1) // CS
        lax.fori_loop(0, nchunks, chunk_body, 0)

        for k in range(2):
            pltpu.sync_copy(
                aggs[k].at[pl.ds(0, NPW * D2)],
                agg_hbms[2 * p + k].at[
                    pl.ds(pl.multiple_of(lo * D2, 8), NPW * D2)])


def _make_agg(D):
    nout = 4 if D > 128 else 2
    D2 = min(D, 128) // 2
    mesh = plsc.VectorSubcoreMesh(core_axis_name="c", subcore_axis_name="s")
    return pl.kernel(
        functools.partial(_agg_body, D),
        out_type=tuple(
            jax.ShapeDtypeStruct((NW * NPW * D2,), jnp.float32)
            for _ in range(nout)),
        mesh=mesh,
        compiler_params=pltpu.CompilerParams(needs_layout_passes=False),
        scratch_types=[
            pltpu.VMEM_SHARED((NW * NPW, 128), jnp.float32),
            pltpu.VMEM(((NPW + 1) * D2,), jnp.float32),
            pltpu.VMEM(((NPW + 1) * D2,), jnp.float32),
            pltpu.VMEM((GB, 128), jnp.float32),
            pltpu.VMEM((GB, 128), jnp.float32),
            pltpu.VMEM((CS,), jnp.int32),
            pltpu.VMEM((CS,), jnp.int32),
            pltpu.VMEM((CS,), jnp.float32),
            pltpu.VMEM((16,), jnp.int32),
            pltpu.SemaphoreType.DMA,
            pltpu.SemaphoreType.DMA,
        ],
    )


def _make_linear_body(nagg):
    def body(*refs):
        x_ref = refs[0]
        a_refs = refs[1:1 + nagg]
        w_ref, b_ref, o_ref = refs[1 + nagg:]
        a = jnp.concatenate([r[...] for r in a_refs], axis=1)
        a = jnp.where(jnp.isneginf(a), 0.0, a)
        x = x_ref[...] + a
        y = jnp.dot(x, w_ref[...], preferred_element_type=jnp.float32)
        o_ref[...] = jnp.maximum(y + b_ref[...], 0.0)
    return body


def _linear(x, aggs, W, b):
    n, din = x.shape
    dk = din // len(aggs)
    dout = W.shape[1]
    bn = 1000
    aspec = pl.BlockSpec((bn, dk), lambda i: (i, 0))
    return pl.pallas_call(
        _make_linear_body(len(aggs)),
        grid=(n // bn,),
        in_specs=[pl.BlockSpec((bn, din), lambda i: (i, 0))]
        + [aspec] * len(aggs)
        + [
            pl.BlockSpec((din, dout), lambda i: (0, 0)),
            pl.BlockSpec((1, dout), lambda i: (0, 0)),
        ],
        out_specs=pl.BlockSpec((bn, dout), lambda i: (i, 0)),
        out_shape=jax.ShapeDtypeStruct((n, dout), jnp.float32),
    )(x, *aggs, W, b.reshape(1, dout))


def kernel(feat, edge_index, edge_weight, W1, b1, W2, b2, W3, b3, W4, b4):
    n = feat.shape[0]
    src = edge_index[0]
    dst = edge_index[1]
    bsrc, bdst, bew, cnt = _make_bucket(src.shape[0])(src, dst, edge_weight)
    h = feat
    for W, b in ((W1, b1), (W2, b2), (W3, b3), (W4, b4)):
        din = W.shape[0]
        hg = jnp.pad(h, ((0, NW * NPW - n), (0, 128 - din if din < 128 else 0)))
        aggs = _make_agg(din)(hg, bsrc, bdst, bew, cnt)
        dk = min(din, 128) // 2
        aggs = [a.reshape(NW * NPW, dk)[:n] for a in aggs]
        h = _linear(h, aggs, W, b)
    return h


# speedup vs baseline: 1.0639x; 1.0012x over previous
import functools

import jax
import jax.numpy as jnp
from jax import lax
from jax.experimental import pallas as pl
from jax.experimental.pallas import tpu as pltpu
from jax.experimental.pallas import tpu_sc as plsc

NC = 2
NS = 16
NW = NC * NS
NPW = 320
DUMMY = NPW
C0 = 4000
CS = 1024
GB = 16
ECAP = 2048 * 80

NEG_INF = float("-inf")


def _worker_id():
    return lax.axis_index("s") * NC + lax.axis_index("c")


def _bucket_body(src_hbm, dst_hbm, ew_hbm, bsrc_hbm, bdst_hbm, bew_hbm,
                 cnt_hbm, srcc, dstc, ewc, csrc, cdst, cew, cntv):
    w = _worker_id()
    lo = w * NPW
    lanes = lax.iota(jnp.int32, 16)
    nchunks = src_hbm.shape[0] // C0

    def chunk_body(chunk, total):
        base = pl.multiple_of(chunk * C0, 8)
        pltpu.sync_copy(src_hbm.at[pl.ds(base, C0)], srcc)
        pltpu.sync_copy(dst_hbm.at[pl.ds(base, C0)], dstc)
        pltpu.sync_copy(ew_hbm.at[pl.ds(base, C0)], ewc)

        def scan_body(i, m):
            off = i * 16
            dv = dstc[pl.ds(off, 16)]
            msk = (dv >= lo) & (dv < lo + NPW)
            cnt16 = jnp.where(msk, 1, 0)
            cum = plsc.cumsum(cnt16)
            pos = m + cum - 1
            plsc.store_scatter(cdst, [pos], dv - lo, mask=msk)
            sv = srcc[pl.ds(off, 16)]
            plsc.store_scatter(csrc, [pos], sv, mask=msk)
            ev = ewc[pl.ds(off, 16)]
            plsc.store_scatter(cew, [pos], ev, mask=msk)
            return m + cum[15]

        m = lax.fori_loop(0, C0 // 16, scan_body, jnp.int32(0))
        cdst[pl.ds(m, 16)] = jnp.full((16,), DUMMY, jnp.int32)
        cdst[pl.ds(m + 16, 16)] = jnp.full((16,), DUMMY, jnp.int32)
        csrc[pl.ds(m, 16)] = jnp.zeros((16,), jnp.int32)
        csrc[pl.ds(m + 16, 16)] = jnp.zeros((16,), jnp.int32)
        cew[pl.ds(m, 16)] = jnp.zeros((16,), jnp.float32)
        cew[pl.ds(m + 16, 16)] = jnp.zeros((16,), jnp.float32)
        m_p = ((m + GB - 1) // GB) * GB

        def flush_body(k, _):
            o = k * 16
            fo = pl.multiple_of(w * ECAP + total + o, 8)
            pltpu.sync_copy(csrc.at[pl.ds(o, 16)], bsrc_hbm.at[pl.ds(fo, 16)])
            pltpu.sync_copy(cdst.at[pl.ds(o, 16)], bdst_hbm.at[pl.ds(fo, 16)])
            pltpu.sync_copy(cew.at[pl.ds(o, 16)], bew_hbm.at[pl.ds(fo, 16)])
            return 0

        lax.fori_loop(0, m_p // 16, flush_body, 0)
        return total + m_p

    total = lax.fori_loop(0, nchunks, chunk_body, jnp.int32(0))
    cntv[...] = jnp.where(lanes == 0, total, 0)
    pltpu.sync_copy(cntv, cnt_hbm.at[pl.ds(pl.multiple_of(w * 16, 8), 16)])


def _make_bucket(E):
    mesh = plsc.VectorSubcoreMesh(core_axis_name="c", subcore_axis_name="s")
    return pl.kernel(
        _bucket_body,
        out_type=(
            jax.ShapeDtypeStruct((NW * ECAP,), jnp.int32),
            jax.ShapeDtypeStruct((NW * ECAP,), jnp.int32),
            jax.ShapeDtypeStruct((NW * ECAP,), jnp.float32),
            jax.ShapeDtypeStruct((NW * 16,), jnp.int32),
        ),
        mesh=mesh,
        compiler_params=pltpu.CompilerParams(needs_layout_passes=False),
        scratch_types=[
            pltpu.VMEM((C0,), jnp.int32),
            pltpu.VMEM((C0,), jnp.int32),
            pltpu.VMEM((C0,), jnp.float32),
            pltpu.VMEM((C0 + GB,), jnp.int32),
            pltpu.VMEM((C0 + GB,), jnp.int32),
            pltpu.VMEM((C0 + GB,), jnp.float32),
            pltpu.VMEM((16,), jnp.int32),
        ],
    )


def _agg_body(D, feat_hbm, bsrc_hbm, bdst_hbm, bew_hbm, cnt_hbm, *rest):
    nout = 4 if D > 128 else 2
    agg_hbms = rest[:nout]
    (shared, agg0, agg1, rows_a, rows_b,
     srcs, dsts, ews, cntv, sem_a, sem_b) = rest[nout:]
    w = _worker_id()
    lo = w * NPW
    s_id = lax.axis_index("s")
    DP = min(D, 128)
    D2 = DP // 2
    npass = max(1, D // 128)
    aggs = (agg0, agg1)

    pltpu.sync_copy(cnt_hbm.at[pl.ds(pl.multiple_of(w * 16, 8), 16)], cntv)
    total = cntv[...][0]

    def fire(g, rows, sem):
        sv = srcs[pl.ds(g * GB, GB)]
        return pltpu.async_copy(shared.at[sv], rows, sem)

    def process(g, rows):
        off = g * GB
        for q in range(GB // 16):
            av = dsts[pl.ds(off + q * 16, 16)] * D2
            ev = ews[pl.ds(off + q * 16, 16)]
            for e in range(16):
                e_s = ev[e]
                abase = av[e]
                for j in range(D2 // 16):
                    for k in range(2):
                        a = aggs[k][pl.ds(abase + j * 16, 16)]
                        r = rows[q * 16 + e, pl.ds(k * D2 + j * 16, 16)]
                        aggs[k][pl.ds(abase + j * 16, 16)] = jnp.maximum(
                            a, r * e_s)

    def chunk_body(c, _):
        cbase = c * CS
        co = pl.multiple_of(w * ECAP + cbase, 8)
        pltpu.sync_copy(bsrc_hbm.at[pl.ds(co, CS)], srcs)
        pltpu.sync_copy(bdst_hbm.at[pl.ds(co, CS)], dsts)
        pltpu.sync_copy(bew_hbm.at[pl.ds(co, CS)], ews)
        nb = jnp.minimum(CS, total - cbase) // GB

        @pl.when(nb > 0)
        def _():
            fire(0, rows_a, sem_a)

        def pair_body(p, _):
            ga = 2 * p
            gb = 2 * p + 1

            @pl.when(gb < nb)
            def _():
                fire(gb, rows_b, sem_b)

            pltpu.make_async_copy(shared.at[srcs[pl.ds(0, GB)]],
                                  rows_a, sem_a).wait()
            process(ga, rows_a)

            @pl.when(ga + 2 < nb)
            def _():
                fire(ga + 2, rows_a, sem_a)

            @pl.when(gb < nb)
            def _():
                pltpu.make_async_copy(shared.at[srcs[pl.ds(0, GB)]],
                                      rows_b, sem_b).wait()
                process(gb, rows_b)

            return 0

        lax.fori_loop(0, (nb + 1) // 2, pair_body, 0)
        return 0

    ro = pl.multiple_of(s_id * (NW * NPW // NS), 8)
    nrs = NW * NPW // NS
    for p in range(npass):
        if p > 0:
            plsc.subcore_barrier()
        if D > 128:
            pltpu.sync_copy(feat_hbm.at[pl.ds(ro, nrs), pl.ds(128 * p, 128)],
                            shared.at[pl.ds(ro, nrs)])
        else:
            pltpu.sync_copy(feat_hbm.at[pl.ds(ro, nrs)],
                            shared.at[pl.ds(ro, nrs)])
        plsc.subcore_barrier()

        def init_body(i, _):
            ninf = jnp.full((16,), NEG_INF, jnp.float32)
            for k in range(2):
                aggs[k][pl.ds(i * 16, 16)] = ninf
            return 0

        lax.fori_loop(0, (NPW + 1) * D2 // 16, init_body, 0)

        nchunks = (total + CS - 1) // CS
        lax.fori_loop(0, nchunks, chunk_body, 0)

        for k in range(2):
            pltpu.sync_copy(
                aggs[k].at[pl.ds(0, NPW * D2)],
                agg_hbms[2 * p + k].at[
                    pl.ds(pl.multiple_of(lo * D2, 8), NPW * D2)])


def _make_agg(D):
    nout = 4 if D > 128 else 2
    D2 = min(D, 128) // 2
    mesh = plsc.VectorSubcoreMesh(core_axis_name="c", subcore_axis_name="s")
    return pl.kernel(
        functools.partial(_agg_body, D),
        out_type=tuple(
            jax.ShapeDtypeStruct((NW * NPW * D2,), jnp.float32)
            for _ in range(nout)),
        mesh=mesh,
        compiler_params=pltpu.CompilerParams(needs_layout_passes=False),
        scratch_types=[
            pltpu.VMEM_SHARED((NW * NPW, 128), jnp.float32),
            pltpu.VMEM(((NPW + 1) * D2,), jnp.float32),
            pltpu.VMEM(((NPW + 1) * D2,), jnp.float32),
            pltpu.VMEM((GB, 128), jnp.float32),
            pltpu.VMEM((GB, 128), jnp.float32),
            pltpu.VMEM((CS,), jnp.int32),
            pltpu.VMEM((CS,), jnp.int32),
            pltpu.VMEM((CS,), jnp.float32),
            pltpu.VMEM((16,), jnp.int32),
            pltpu.SemaphoreType.DMA,
            pltpu.SemaphoreType.DMA,
        ],
    )


def _make_linear_body(nagg):
    def body(*refs):
        x_ref = refs[0]
        a_refs = refs[1:1 + nagg]
        w_ref, b_ref, o_ref = refs[1 + nagg:]
        a = jnp.concatenate([r[...] for r in a_refs], axis=1)
        a = jnp.where(jnp.isneginf(a), 0.0, a)
        x = x_ref[...] + a
        y = jnp.dot(x, w_ref[...], preferred_element_type=jnp.float32)
        o_ref[...] = jnp.maximum(y + b_ref[...], 0.0)
    return body


def _linear(x, aggs, W, b):
    n, din = x.shape
    dk = din // len(aggs)
    dout = W.shape[1]
    bn = 1000
    aspec = pl.BlockSpec((bn, dk), lambda i: (i, 0))
    return pl.pallas_call(
        _make_linear_body(len(aggs)),
        grid=(n // bn,),
        in_specs=[pl.BlockSpec((bn, din), lambda i: (i, 0))]
        + [aspec] * len(aggs)
        + [
            pl.BlockSpec((din, dout), lambda i: (0, 0)),
            pl.BlockSpec((1, dout), lambda i: (0, 0)),
        ],
        out_specs=pl.BlockSpec((bn, dout), lambda i: (i, 0)),
        out_shape=jax.ShapeDtypeStruct((n, dout), jnp.float32),
    )(x, *aggs, W, b.reshape(1, dout))


def kernel(feat, edge_index, edge_weight, W1, b1, W2, b2, W3, b3, W4, b4):
    n = feat.shape[0]
    src = edge_index[0]
    dst = edge_index[1]
    bsrc, bdst, bew, cnt = _make_bucket(src.shape[0])(src, dst, edge_weight)
    h = feat
    for W, b in ((W1, b1), (W2, b2), (W3, b3), (W4, b4)):
        din = W.shape[0]
        hg = jnp.pad(h, ((0, NW * NPW - n), (0, 128 - din if din < 128 else 0)))
        aggs = _make_agg(din)(hg, bsrc, bdst, bew, cnt)
        dk = min(din, 128) // 2
        aggs = [a.reshape(NW * NPW, dk)[:n] for a in aggs]
        h = _linear(h, aggs, W, b)
    return h


# speedup vs baseline: 1.0858x; 1.0206x over previous
import functools

import jax
import jax.numpy as jnp
from jax import lax
from jax.experimental import pallas as pl
from jax.experimental.pallas import tpu as pltpu
from jax.experimental.pallas import tpu_sc as plsc

NC = 2
NS = 16
NW = NC * NS
NPW = 320
DUMMY = NPW
C0 = 4000
CS = 1024
GB = 16
ECAP = 2048 * 80

NEG_INF = float("-inf")


def _worker_id():
    return lax.axis_index("s") * NC + lax.axis_index("c")


def _bucket_body(src_hbm, dst_hbm, ew_hbm, bsrc_hbm, bdst_hbm, bew_hbm,
                 cnt_hbm, srcc, dstc, ewc, csrc, cdst, cew, cntv):
    w = _worker_id()
    lo = w * NPW
    lanes = lax.iota(jnp.int32, 16)
    nchunks = src_hbm.shape[0] // C0

    def chunk_body(chunk, total):
        base = pl.multiple_of(chunk * C0, 8)
        pltpu.sync_copy(src_hbm.at[pl.ds(base, C0)], srcc)
        pltpu.sync_copy(dst_hbm.at[pl.ds(base, C0)], dstc)
        pltpu.sync_copy(ew_hbm.at[pl.ds(base, C0)], ewc)

        def scan_body(i, m):
            off = i * 32
            dv1 = dstc[pl.ds(off, 16)]
            dv2 = dstc[pl.ds(off + 16, 16)]
            msk1 = (dv1 >= lo) & (dv1 < lo + NPW)
            msk2 = (dv2 >= lo) & (dv2 < lo + NPW)
            c1 = plsc.cumsum(jnp.where(msk1, 1, 0))
            c2 = plsc.cumsum(jnp.where(msk2, 1, 0))
            t1 = c1[15]
            pos1 = m + c1 - 1
            pos2 = m + t1 + c2 - 1
            plsc.store_scatter(cdst, [pos1], dv1 - lo, mask=msk1)
            plsc.store_scatter(cdst, [pos2], dv2 - lo, mask=msk2)
            sv1 = srcc[pl.ds(off, 16)]
            sv2 = srcc[pl.ds(off + 16, 16)]
            plsc.store_scatter(csrc, [pos1], sv1, mask=msk1)
            plsc.store_scatter(csrc, [pos2], sv2, mask=msk2)
            ev1 = ewc[pl.ds(off, 16)]
            ev2 = ewc[pl.ds(off + 16, 16)]
            plsc.store_scatter(cew, [pos1], ev1, mask=msk1)
            plsc.store_scatter(cew, [pos2], ev2, mask=msk2)
            return m + t1 + c2[15]

        m = lax.fori_loop(0, C0 // 32, scan_body, jnp.int32(0))
        cdst[pl.ds(m, 16)] = jnp.full((16,), DUMMY, jnp.int32)
        cdst[pl.ds(m + 16, 16)] = jnp.full((16,), DUMMY, jnp.int32)
        csrc[pl.ds(m, 16)] = jnp.zeros((16,), jnp.int32)
        csrc[pl.ds(m + 16, 16)] = jnp.zeros((16,), jnp.int32)
        cew[pl.ds(m, 16)] = jnp.zeros((16,), jnp.float32)
        cew[pl.ds(m + 16, 16)] = jnp.zeros((16,), jnp.float32)
        m_p = ((m + GB - 1) // GB) * GB

        def flush_body(k, _):
            o = k * 16
            fo = pl.multiple_of(w * ECAP + total + o, 8)
            pltpu.sync_copy(csrc.at[pl.ds(o, 16)], bsrc_hbm.at[pl.ds(fo, 16)])
            pltpu.sync_copy(cdst.at[pl.ds(o, 16)], bdst_hbm.at[pl.ds(fo, 16)])
            pltpu.sync_copy(cew.at[pl.ds(o, 16)], bew_hbm.at[pl.ds(fo, 16)])
            return 0

        lax.fori_loop(0, m_p // 16, flush_body, 0)
        return total + m_p

    total = lax.fori_loop(0, nchunks, chunk_body, jnp.int32(0))
    cntv[...] = jnp.where(lanes == 0, total, 0)
    pltpu.sync_copy(cntv, cnt_hbm.at[pl.ds(pl.multiple_of(w * 16, 8), 16)])


def _make_bucket(E):
    mesh = plsc.VectorSubcoreMesh(core_axis_name="c", subcore_axis_name="s")
    return pl.kernel(
        _bucket_body,
        out_type=(
            jax.ShapeDtypeStruct((NW * ECAP,), jnp.int32),
            jax.ShapeDtypeStruct((NW * ECAP,), jnp.int32),
            jax.ShapeDtypeStruct((NW * ECAP,), jnp.float32),
            jax.ShapeDtypeStruct((NW * 16,), jnp.int32),
        ),
        mesh=mesh,
        compiler_params=pltpu.CompilerParams(needs_layout_passes=False),
        scratch_types=[
            pltpu.VMEM((C0,), jnp.int32),
            pltpu.VMEM((C0,), jnp.int32),
            pltpu.VMEM((C0,), jnp.float32),
            pltpu.VMEM((C0 + GB,), jnp.int32),
            pltpu.VMEM((C0 + GB,), jnp.int32),
            pltpu.VMEM((C0 + GB,), jnp.float32),
            pltpu.VMEM((16,), jnp.int32),
        ],
    )


def _agg_body(D, feat_hbm, bsrc_hbm, bdst_hbm, bew_hbm, cnt_hbm, *rest):
    nout = 4 if D > 128 else 2
    agg_hbms = rest[:nout]
    (shared, agg0, agg1, rows_a, rows_b,
     srcs, dsts, ews, cntv, sem_a, sem_b) = rest[nout:]
    w = _worker_id()
    lo = w * NPW
    s_id = lax.axis_index("s")
    DP = min(D, 128)
    D2 = DP // 2
    npass = max(1, D // 128)
    aggs = (agg0, agg1)

    pltpu.sync_copy(cnt_hbm.at[pl.ds(pl.multiple_of(w * 16, 8), 16)], cntv)
    total = cntv[...][0]

    def fire(g, rows, sem):
        sv = srcs[pl.ds(g * GB, GB)]
        return pltpu.async_copy(shared.at[sv], rows, sem)

    def process(g, rows):
        off = g * GB
        for q in range(GB // 16):
            av = dsts[pl.ds(off + q * 16, 16)] * D2
            ev = ews[pl.ds(off + q * 16, 16)]
            for e in range(16):
                e_s = ev[e]
                abase = av[e]
                for j in range(D2 // 16):
                    for k in range(2):
                        a = aggs[k][pl.ds(abase + j * 16, 16)]
                        r = rows[q * 16 + e, pl.ds(k * D2 + j * 16, 16)]
                        aggs[k][pl.ds(abase + j * 16, 16)] = jnp.maximum(
                            a, r * e_s)

    def chunk_body(c, _):
        cbase = c * CS
        co = pl.multiple_of(w * ECAP + cbase, 8)
        pltpu.sync_copy(bsrc_hbm.at[pl.ds(co, CS)], srcs)
        pltpu.sync_copy(bdst_hbm.at[pl.ds(co, CS)], dsts)
        pltpu.sync_copy(bew_hbm.at[pl.ds(co, CS)], ews)
        nb = jnp.minimum(CS, total - cbase) // GB

        @pl.when(nb > 0)
        def _():
            fire(0, rows_a, sem_a)

        def pair_body(p, _):
            ga = 2 * p
            gb = 2 * p + 1

            @pl.when(gb < nb)
            def _():
                fire(gb, rows_b, sem_b)

            pltpu.make_async_copy(shared.at[srcs[pl.ds(0, GB)]],
                                  rows_a, sem_a).wait()
            process(ga, rows_a)

            @pl.when(ga + 2 < nb)
            def _():
                fire(ga + 2, rows_a, sem_a)

            @pl.when(gb < nb)
            def _():
                pltpu.make_async_copy(shared.at[srcs[pl.ds(0, GB)]],
                                      rows_b, sem_b).wait()
                process(gb, rows_b)

            return 0

        lax.fori_loop(0, (nb + 1) // 2, pair_body, 0)
        return 0

    ro = pl.multiple_of(s_id * (NW * NPW // NS), 8)
    nrs = NW * NPW // NS
    for p in range(npass):
        if p > 0:
            plsc.subcore_barrier()
        if D > 128:
            pltpu.sync_copy(feat_hbm.at[pl.ds(ro, nrs), pl.ds(128 * p, 128)],
                            shared.at[pl.ds(ro, nrs)])
        else:
            pltpu.sync_copy(feat_hbm.at[pl.ds(ro, nrs)],
                            shared.at[pl.ds(ro, nrs)])
        plsc.subcore_barrier()

        def init_body(i, _):
            ninf = jnp.full((16,), NEG_INF, jnp.float32)
            for k in range(2):
                aggs[k][pl.ds(i * 16, 16)] = ninf
            return 0

        lax.fori_loop(0, (NPW + 1) * D2 // 16, init_body, 0)

        nchunks = (total + CS - 1) // CS
        lax.fori_loop(0, nchunks, chunk_body, 0)

        for k in range(2):
            pltpu.sync_copy(
                aggs[k].at[pl.ds(0, NPW * D2)],
                agg_hbms[2 * p + k].at[
                    pl.ds(pl.multiple_of(lo * D2, 8), NPW * D2)])


def _make_agg(D):
    nout = 4 if D > 128 else 2
    D2 = min(D, 128) // 2
    mesh = plsc.VectorSubcoreMesh(core_axis_name="c", subcore_axis_name="s")
    return pl.kernel(
        functools.partial(_agg_body, D),
        out_type=tuple(
            jax.ShapeDtypeStruct((NW * NPW * D2,), jnp.float32)
            for _ in range(nout)),
        mesh=mesh,
        compiler_params=pltpu.CompilerParams(needs_layout_passes=False),
        scratch_types=[
            pltpu.VMEM_SHARED((NW * NPW, 128), jnp.float32),
            pltpu.VMEM(((NPW + 1) * D2,), jnp.float32),
            pltpu.VMEM(((NPW + 1) * D2,), jnp.float32),
            pltpu.VMEM((GB, 128), jnp.float32),
            pltpu.VMEM((GB, 128), jnp.float32),
            pltpu.VMEM((CS,), jnp.int32),
            pltpu.VMEM((CS,), jnp.int32),
            pltpu.VMEM((CS,), jnp.float32),
            pltpu.VMEM((16,), jnp.int32),
            pltpu.SemaphoreType.DMA,
            pltpu.SemaphoreType.DMA,
        ],
    )


def _make_linear_body(nagg):
    def body(*refs):
        x_ref = refs[0]
        a_refs = refs[1:1 + nagg]
        w_ref, b_ref, o_ref = refs[1 + nagg:]
        a = jnp.concatenate([r[...] for r in a_refs], axis=1)
        a = jnp.where(jnp.isneginf(a), 0.0, a)
        x = x_ref[...] + a
        y = jnp.dot(x, w_ref[...], preferred_element_type=jnp.float32)
        o_ref[...] = jnp.maximum(y + b_ref[...], 0.0)
    return body


def _linear(x, aggs, W, b):
    n, din = x.shape
    dk = din // len(aggs)
    dout = W.shape[1]
    bn = 1000
    aspec = pl.BlockSpec((bn, dk), lambda i: (i, 0))
    return pl.pallas_call(
        _make_linear_body(len(aggs)),
        grid=(n // bn,),
        in_specs=[pl.BlockSpec((bn, din), lambda i: (i, 0))]
        + [aspec] * len(aggs)
        + [
            pl.BlockSpec((din, dout), lambda i: (0, 0)),
            pl.BlockSpec((1, dout), lambda i: (0, 0)),
        ],
        out_specs=pl.BlockSpec((bn, dout), lambda i: (i, 0)),
        out_shape=jax.ShapeDtypeStruct((n, dout), jnp.float32),
    )(x, *aggs, W, b.reshape(1, dout))


def kernel(feat, edge_index, edge_weight, W1, b1, W2, b2, W3, b3, W4, b4):
    n = feat.shape[0]
    src = edge_index[0]
    dst = edge_index[1]
    bsrc, bdst, bew, cnt = _make_bucket(src.shape[0])(src, dst, edge_weight)
    h = feat
    for W, b in ((W1, b1), (W2, b2), (W3, b3), (W4, b4)):
        din = W.shape[0]
        hg = jnp.pad(h, ((0, NW * NPW - n), (0, 128 - din if din < 128 else 0)))
        aggs = _make_agg(din)(hg, bsrc, bdst, bew, cnt)
        dk = min(din, 128) // 2
        aggs = [a.reshape(NW * NPW, dk)[:n] for a in aggs]
        h = _linear(h, aggs, W, b)
    return h


# speedup vs baseline: 1.1327x; 1.0432x over previous
import functools

import jax
import jax.numpy as jnp
from jax import lax
from jax.experimental import pallas as pl
from jax.experimental.pallas import tpu as pltpu
from jax.experimental.pallas import tpu_sc as plsc

NC = 2
NS = 16
NW = NC * NS
NPW = 320
DUMMY = NPW
C0 = 8000
CS = 1024
GB = 16
ECAP = 2048 * 80

NEG_INF = float("-inf")


def _worker_id():
    return lax.axis_index("s") * NC + lax.axis_index("c")


def _bucket_body(src_hbm, dst_hbm, ew_hbm, bsrc_hbm, bdst_hbm, bew_hbm,
                 cnt_hbm, srcc, dstc, ewc, csrc, cdst, cew, cntv):
    w = _worker_id()
    lo = w * NPW
    lanes = lax.iota(jnp.int32, 16)
    nchunks = src_hbm.shape[0] // C0

    def chunk_body(chunk, total):
        base = pl.multiple_of(chunk * C0, 8)
        pltpu.sync_copy(src_hbm.at[pl.ds(base, C0)], srcc)
        pltpu.sync_copy(dst_hbm.at[pl.ds(base, C0)], dstc)
        pltpu.sync_copy(ew_hbm.at[pl.ds(base, C0)], ewc)

        def scan_body(i, m):
            off = i * 32
            dv1 = dstc[pl.ds(off, 16)]
            dv2 = dstc[pl.ds(off + 16, 16)]
            msk1 = (dv1 >= lo) & (dv1 < lo + NPW)
            msk2 = (dv2 >= lo) & (dv2 < lo + NPW)
            c1 = plsc.cumsum(jnp.where(msk1, 1, 0))
            c2 = plsc.cumsum(jnp.where(msk2, 1, 0))
            t1 = c1[15]
            pos1 = m + c1 - 1
            pos2 = m + t1 + c2 - 1
            plsc.store_scatter(cdst, [pos1], dv1 - lo, mask=msk1)
            plsc.store_scatter(cdst, [pos2], dv2 - lo, mask=msk2)
            sv1 = srcc[pl.ds(off, 16)]
            sv2 = srcc[pl.ds(off + 16, 16)]
            plsc.store_scatter(csrc, [pos1], sv1, mask=msk1)
            plsc.store_scatter(csrc, [pos2], sv2, mask=msk2)
            ev1 = ewc[pl.ds(off, 16)]
            ev2 = ewc[pl.ds(off + 16, 16)]
            plsc.store_scatter(cew, [pos1], ev1, mask=msk1)
            plsc.store_scatter(cew, [pos2], ev2, mask=msk2)
            return m + t1 + c2[15]

        m = lax.fori_loop(0, C0 // 32, scan_body, jnp.int32(0))
        cdst[pl.ds(m, 16)] = jnp.full((16,), DUMMY, jnp.int32)
        cdst[pl.ds(m + 16, 16)] = jnp.full((16,), DUMMY, jnp.int32)
        csrc[pl.ds(m, 16)] = jnp.zeros((16,), jnp.int32)
        csrc[pl.ds(m + 16, 16)] = jnp.zeros((16,), jnp.int32)
        cew[pl.ds(m, 16)] = jnp.zeros((16,), jnp.float32)
        cew[pl.ds(m + 16, 16)] = jnp.zeros((16,), jnp.float32)
        m_p = ((m + GB - 1) // GB) * GB

        def flush_body(k, _):
            o = k * 16
            fo = pl.multiple_of(w * ECAP + total + o, 8)
            pltpu.sync_copy(csrc.at[pl.ds(o, 16)], bsrc_hbm.at[pl.ds(fo, 16)])
            pltpu.sync_copy(cdst.at[pl.ds(o, 16)], bdst_hbm.at[pl.ds(fo, 16)])
            pltpu.sync_copy(cew.at[pl.ds(o, 16)], bew_hbm.at[pl.ds(fo, 16)])
            return 0

        lax.fori_loop(0, m_p // 16, flush_body, 0)
        return total + m_p

    total = lax.fori_loop(0, nchunks, chunk_body, jnp.int32(0))
    cntv[...] = jnp.where(lanes == 0, total, 0)
    pltpu.sync_copy(cntv, cnt_hbm.at[pl.ds(pl.multiple_of(w * 16, 8), 16)])


def _make_bucket(E):
    mesh = plsc.VectorSubcoreMesh(core_axis_name="c", subcore_axis_name="s")
    return pl.kernel(
        _bucket_body,
        out_type=(
            jax.ShapeDtypeStruct((NW * ECAP,), jnp.int32),
            jax.ShapeDtypeStruct((NW * ECAP,), jnp.int32),
            jax.ShapeDtypeStruct((NW * ECAP,), jnp.float32),
            jax.ShapeDtypeStruct((NW * 16,), jnp.int32),
        ),
        mesh=mesh,
        compiler_params=pltpu.CompilerParams(needs_layout_passes=False),
        scratch_types=[
            pltpu.VMEM((C0,), jnp.int32),
            pltpu.VMEM((C0,), jnp.int32),
            pltpu.VMEM((C0,), jnp.float32),
            pltpu.VMEM((C0 + GB,), jnp.int32),
            pltpu.VMEM((C0 + GB,), jnp.int32),
            pltpu.VMEM((C0 + GB,), jnp.float32),
            pltpu.VMEM((16,), jnp.int32),
        ],
    )


def _agg_body(D, feat_hbm, bsrc_hbm, bdst_hbm, bew_hbm, cnt_hbm, *rest):
    nout = 4 if D > 128 else 2
    agg_hbms = rest[:nout]
    (shared, agg0, agg1, rows_a, rows_b,
     srcs, dsts, ews, cntv, sem_a, sem_b) = rest[nout:]
    w = _worker_id()
    lo = w * NPW
    s_id = lax.axis_index("s")
    DP = min(D, 128)
    D2 = DP // 2
    npass = max(1, D // 128)
    aggs = (agg0, agg1)

    pltpu.sync_copy(cnt_hbm.at[pl.ds(pl.multiple_of(w * 16, 8), 16)], cntv)
    total = cntv[...][0]

    def fire(g, rows, sem):
        sv = srcs[pl.ds(g * GB, GB)]
        return pltpu.async_copy(shared.at[sv], rows, sem)

    def process(g, rows):
        off = g * GB
        for q in range(GB // 16):
            av = dsts[pl.ds(off + q * 16, 16)] * D2
            ev = ews[pl.ds(off + q * 16, 16)]
            for e in range(16):
                e_s = ev[e]
                abase = av[e]
                for j in range(D2 // 16):
                    for k in range(2):
                        a = aggs[k][pl.ds(abase + j * 16, 16)]
                        r = rows[q * 16 + e, pl.ds(k * D2 + j * 16, 16)]
                        aggs[k][pl.ds(abase + j * 16, 16)] = jnp.maximum(
                            a, r * e_s)

    def chunk_body(c, _):
        cbase = c * CS
        co = pl.multiple_of(w * ECAP + cbase, 8)
        pltpu.sync_copy(bsrc_hbm.at[pl.ds(co, CS)], srcs)
        pltpu.sync_copy(bdst_hbm.at[pl.ds(co, CS)], dsts)
        pltpu.sync_copy(bew_hbm.at[pl.ds(co, CS)], ews)
        nb = jnp.minimum(CS, total - cbase) // GB

        @pl.when(nb > 0)
        def _():
            fire(0, rows_a, sem_a)

        def pair_body(p, _):
            ga = 2 * p
            gb = 2 * p + 1

            @pl.when(gb < nb)
            def _():
                fire(gb, rows_b, sem_b)

            pltpu.make_async_copy(shared.at[srcs[pl.ds(0, GB)]],
                                  rows_a, sem_a).wait()
            process(ga, rows_a)

            @pl.when(ga + 2 < nb)
            def _():
                fire(ga + 2, rows_a, sem_a)

            @pl.when(gb < nb)
            def _():
                pltpu.make_async_copy(shared.at[srcs[pl.ds(0, GB)]],
                                      rows_b, sem_b).wait()
                process(gb, rows_b)

            return 0

        lax.fori_loop(0, (nb + 1) // 2, pair_body, 0)
        return 0

    ro = pl.multiple_of(s_id * (NW * NPW // NS), 8)
    nrs = NW * NPW // NS
    for p in range(npass):
        if p > 0:
            plsc.subcore_barrier()
        if D > 128:
            pltpu.sync_copy(feat_hbm.at[pl.ds(ro, nrs), pl.ds(128 * p, 128)],
                            shared.at[pl.ds(ro, nrs)])
        else:
            pltpu.sync_copy(feat_hbm.at[pl.ds(ro, nrs)],
                            shared.at[pl.ds(ro, nrs)])
        plsc.subcore_barrier()

        def init_body(i, _):
            ninf = jnp.full((16,), NEG_INF, jnp.float32)
            for k in range(2):
                aggs[k][pl.ds(i * 16, 16)] = ninf
            return 0

        lax.fori_loop(0, (NPW + 1) * D2 // 16, init_body, 0)

        nchunks = (total + CS - 1) // CS
        lax.fori_loop(0, nchunks, chunk_body, 0)

        for k in range(2):
            pltpu.sync_copy(
                aggs[k].at[pl.ds(0, NPW * D2)],
                agg_hbms[2 * p + k].at[
                    pl.ds(pl.multiple_of(lo * D2, 8), NPW * D2)])


def _make_agg(D):
    nout = 4 if D > 128 else 2
    D2 = min(D, 128) // 2
    mesh = plsc.VectorSubcoreMesh(core_axis_name="c", subcore_axis_name="s")
    return pl.kernel(
        functools.partial(_agg_body, D),
        out_type=tuple(
            jax.ShapeDtypeStruct((NW * NPW * D2,), jnp.float32)
            for _ in range(nout)),
        mesh=mesh,
        compiler_params=pltpu.CompilerParams(needs_layout_passes=False),
        scratch_types=[
            pltpu.VMEM_SHARED((NW * NPW, 128), jnp.float32),
            pltpu.VMEM(((NPW + 1) * D2,), jnp.float32),
            pltpu.VMEM(((NPW + 1) * D2,), jnp.float32),
            pltpu.VMEM((GB, 128), jnp.float32),
            pltpu.VMEM((GB, 128), jnp.float32),
            pltpu.VMEM((CS,), jnp.int32),
            pltpu.VMEM((CS,), jnp.int32),
            pltpu.VMEM((CS,), jnp.float32),
            pltpu.VMEM((16,), jnp.int32),
            pltpu.SemaphoreType.DMA,
            pltpu.SemaphoreType.DMA,
        ],
    )


def _make_linear_body(nagg):
    def body(*refs):
        x_ref = refs[0]
        a_refs = refs[1:1 + nagg]
        w_ref, b_ref, o_ref = refs[1 + nagg:]
        a = jnp.concatenate([r[...] for r in a_refs], axis=1)
        a = jnp.where(jnp.isneginf(a), 0.0, a)
        x = x_ref[...] + a
        y = jnp.dot(x, w_ref[...], preferred_element_type=jnp.float32)
        o_ref[...] = jnp.maximum(y + b_ref[...], 0.0)
    return body


def _linear(x, aggs, W, b):
    n, din = x.shape
    dk = din // len(aggs)
    dout = W.shape[1]
    bn = 1000
    aspec = pl.BlockSpec((bn, dk), lambda i: (i, 0))
    return pl.pallas_call(
        _make_linear_body(len(aggs)),
        grid=(n // bn,),
        in_specs=[pl.BlockSpec((bn, din), lambda i: (i, 0))]
        + [aspec] * len(aggs)
        + [
            pl.BlockSpec((din, dout), lambda i: (0, 0)),
            pl.BlockSpec((1, dout), lambda i: (0, 0)),
        ],
        out_specs=pl.BlockSpec((bn, dout), lambda i: (i, 0)),
        out_shape=jax.ShapeDtypeStruct((n, dout), jnp.float32),
    )(x, *aggs, W, b.reshape(1, dout))


def kernel(feat, edge_index, edge_weight, W1, b1, W2, b2, W3, b3, W4, b4):
    n = feat.shape[0]
    src = edge_index[0]
    dst = edge_index[1]
    bsrc, bdst, bew, cnt = _make_bucket(src.shape[0])(src, dst, edge_weight)
    h = feat
    for W, b in ((W1, b1), (W2, b2), (W3, b3), (W4, b4)):
        din = W.shape[0]
        hg = jnp.pad(h, ((0, NW * NPW - n), (0, 128 - din if din < 128 else 0)))
        aggs = _make_agg(din)(hg, bsrc, bdst, bew, cnt)
        dk = min(din, 128) // 2
        aggs = [a.reshape(NW * NPW, dk)[:n] for a in aggs]
        h = _linear(h, aggs, W, b)
    return h


# speedup vs baseline: 1.1542x; 1.0190x over previous
import functools

import jax
import jax.numpy as jnp
from jax import lax
from jax.experimental import pallas as pl
from jax.experimental.pallas import tpu as pltpu
from jax.experimental.pallas import tpu_sc as plsc

NC = 2
NS = 16
NW = NC * NS
NPW = 320
DUMMY = NPW
C0 = 16000
CS = 1024
GB = 16
ECAP = 2048 * 80

NEG_INF = float("-inf")


def _worker_id():
    return lax.axis_index("s") * NC + lax.axis_index("c")


def _bucket_body(src_hbm, dst_hbm, ew_hbm, bsrc_hbm, bdst_hbm, bew_hbm,
                 cnt_hbm, srcc, dstc, ewc, csrc, cdst, cew, cntv):
    w = _worker_id()
    lo = w * NPW
    lanes = lax.iota(jnp.int32, 16)
    nchunks = src_hbm.shape[0] // C0

    def chunk_body(chunk, total):
        base = pl.multiple_of(chunk * C0, 8)
        pltpu.sync_copy(src_hbm.at[pl.ds(base, C0)], srcc)
        pltpu.sync_copy(dst_hbm.at[pl.ds(base, C0)], dstc)
        pltpu.sync_copy(ew_hbm.at[pl.ds(base, C0)], ewc)

        def scan_body(i, m):
            off = i * 32
            dv1 = dstc[pl.ds(off, 16)]
            dv2 = dstc[pl.ds(off + 16, 16)]
            msk1 = (dv1 >= lo) & (dv1 < lo + NPW)
            msk2 = (dv2 >= lo) & (dv2 < lo + NPW)
            c1 = plsc.cumsum(jnp.where(msk1, 1, 0))
            c2 = plsc.cumsum(jnp.where(msk2, 1, 0))
            t1 = c1[15]
            pos1 = m + c1 - 1
            pos2 = m + t1 + c2 - 1
            plsc.store_scatter(cdst, [pos1], dv1 - lo, mask=msk1)
            plsc.store_scatter(cdst, [pos2], dv2 - lo, mask=msk2)
            sv1 = srcc[pl.ds(off, 16)]
            sv2 = srcc[pl.ds(off + 16, 16)]
            plsc.store_scatter(csrc, [pos1], sv1, mask=msk1)
            plsc.store_scatter(csrc, [pos2], sv2, mask=msk2)
            ev1 = ewc[pl.ds(off, 16)]
            ev2 = ewc[pl.ds(off + 16, 16)]
            plsc.store_scatter(cew, [pos1], ev1, mask=msk1)
            plsc.store_scatter(cew, [pos2], ev2, mask=msk2)
            return m + t1 + c2[15]

        m = lax.fori_loop(0, C0 // 32, scan_body, jnp.int32(0))
        cdst[pl.ds(m, 16)] = jnp.full((16,), DUMMY, jnp.int32)
        cdst[pl.ds(m + 16, 16)] = jnp.full((16,), DUMMY, jnp.int32)
        csrc[pl.ds(m, 16)] = jnp.zeros((16,), jnp.int32)
        csrc[pl.ds(m + 16, 16)] = jnp.zeros((16,), jnp.int32)
        cew[pl.ds(m, 16)] = jnp.zeros((16,), jnp.float32)
        cew[pl.ds(m + 16, 16)] = jnp.zeros((16,), jnp.float32)
        m_p = ((m + GB - 1) // GB) * GB

        def flush_body(k, _):
            o = k * 16
            fo = pl.multiple_of(w * ECAP + total + o, 8)
            pltpu.sync_copy(csrc.at[pl.ds(o, 16)], bsrc_hbm.at[pl.ds(fo, 16)])
            pltpu.sync_copy(cdst.at[pl.ds(o, 16)], bdst_hbm.at[pl.ds(fo, 16)])
            pltpu.sync_copy(cew.at[pl.ds(o, 16)], bew_hbm.at[pl.ds(fo, 16)])
            return 0

        lax.fori_loop(0, m_p // 16, flush_body, 0)
        return total + m_p

    total = lax.fori_loop(0, nchunks, chunk_body, jnp.int32(0))
    cntv[...] = jnp.where(lanes == 0, total, 0)
    pltpu.sync_copy(cntv, cnt_hbm.at[pl.ds(pl.multiple_of(w * 16, 8), 16)])


def _make_bucket(E):
    mesh = plsc.VectorSubcoreMesh(core_axis_name="c", subcore_axis_name="s")
    return pl.kernel(
        _bucket_body,
        out_type=(
            jax.ShapeDtypeStruct((NW * ECAP,), jnp.int32),
            jax.ShapeDtypeStruct((NW * ECAP,), jnp.int32),
            jax.ShapeDtypeStruct((NW * ECAP,), jnp.float32),
            jax.ShapeDtypeStruct((NW * 16,), jnp.int32),
        ),
        mesh=mesh,
        compiler_params=pltpu.CompilerParams(needs_layout_passes=False),
        scratch_types=[
            pltpu.VMEM((C0,), jnp.int32),
            pltpu.VMEM((C0,), jnp.int32),
            pltpu.VMEM((C0,), jnp.float32),
            pltpu.VMEM((C0 + GB,), jnp.int32),
            pltpu.VMEM((C0 + GB,), jnp.int32),
            pltpu.VMEM((C0 + GB,), jnp.float32),
            pltpu.VMEM((16,), jnp.int32),
        ],
    )


def _agg_body(D, feat_hbm, bsrc_hbm, bdst_hbm, bew_hbm, cnt_hbm, *rest):
    nout = 4 if D > 128 else 2
    agg_hbms = rest[:nout]
    (shared, agg0, agg1, rows_a, rows_b,
     srcs, dsts, ews, cntv, sem_a, sem_b) = rest[nout:]
    w = _worker_id()
    lo = w * NPW
    s_id = lax.axis_index("s")
    DP = min(D, 128)
    D2 = DP // 2
    npass = max(1, D // 128)
    aggs = (agg0, agg1)

    pltpu.sync_copy(cnt_hbm.at[pl.ds(pl.multiple_of(w * 16, 8), 16)], cntv)
    total = cntv[...][0]

    def fire(g, rows, sem):
        sv = srcs[pl.ds(g * GB, GB)]
        return pltpu.async_copy(shared.at[sv], rows, sem)

    def process(g, rows):
        off = g * GB
        for q in range(GB // 16):
            av = dsts[pl.ds(off + q * 16, 16)] * D2
            ev = ews[pl.ds(off + q * 16, 16)]
            for e in range(16):
                e_s = ev[e]
                abase = av[e]
                for j in range(D2 // 16):
                    for k in range(2):
                        a = aggs[k][pl.ds(abase + j * 16, 16)]
                        r = rows[q * 16 + e, pl.ds(k * D2 + j * 16, 16)]
                        aggs[k][pl.ds(abase + j * 16, 16)] = jnp.maximum(
                            a, r * e_s)

    def chunk_body(c, _):
        cbase = c * CS
        co = pl.multiple_of(w * ECAP + cbase, 8)
        pltpu.sync_copy(bsrc_hbm.at[pl.ds(co, CS)], srcs)
        pltpu.sync_copy(bdst_hbm.at[pl.ds(co, CS)], dsts)
        pltpu.sync_copy(bew_hbm.at[pl.ds(co, CS)], ews)
        nb = jnp.minimum(CS, total - cbase) // GB

        @pl.when(nb > 0)
        def _():
            fire(0, rows_a, sem_a)

        def pair_body(p, _):
            ga = 2 * p
            gb = 2 * p + 1

            @pl.when(gb < nb)
            def _():
                fire(gb, rows_b, sem_b)

            pltpu.make_async_copy(shared.at[srcs[pl.ds(0, GB)]],
                                  rows_a, sem_a).wait()
            process(ga, rows_a)

            @pl.when(ga + 2 < nb)
            def _():
                fire(ga + 2, rows_a, sem_a)

            @pl.when(gb < nb)
            def _():
                pltpu.make_async_copy(shared.at[srcs[pl.ds(0, GB)]],
                                      rows_b, sem_b).wait()
                process(gb, rows_b)

            return 0

        lax.fori_loop(0, (nb + 1) // 2, pair_body, 0)
        return 0

    ro = pl.multiple_of(s_id * (NW * NPW // NS), 8)
    nrs = NW * NPW // NS
    for p in range(npass):
        if p > 0:
            plsc.subcore_barrier()
        if D > 128:
            pltpu.sync_copy(feat_hbm.at[pl.ds(ro, nrs), pl.ds(128 * p, 128)],
                            shared.at[pl.ds(ro, nrs)])
        else:
            pltpu.sync_copy(feat_hbm.at[pl.ds(ro, nrs)],
                            shared.at[pl.ds(ro, nrs)])
        plsc.subcore_barrier()

        def init_body(i, _):
            ninf = jnp.full((16,), NEG_INF, jnp.float32)
            for k in range(2):
                aggs[k][pl.ds(i * 16, 16)] = ninf
            return 0

        lax.fori_loop(0, (NPW + 1) * D2 // 16, init_body, 0)

        nchunks = (total + CS - 1) // CS
        lax.fori_loop(0, nchunks, chunk_body, 0)

        for k in range(2):
            pltpu.sync_copy(
                aggs[k].at[pl.ds(0, NPW * D2)],
                agg_hbms[2 * p + k].at[
                    pl.ds(pl.multiple_of(lo * D2, 8), NPW * D2)])


def _make_agg(D):
    nout = 4 if D > 128 else 2
    D2 = min(D, 128) // 2
    mesh = plsc.VectorSubcoreMesh(core_axis_name="c", subcore_axis_name="s")
    return pl.kernel(
        functools.partial(_agg_body, D),
        out_type=tuple(
            jax.ShapeDtypeStruct((NW * NPW * D2,), jnp.float32)
            for _ in range(nout)),
        mesh=mesh,
        compiler_params=pltpu.CompilerParams(needs_layout_passes=False),
        scratch_types=[
            pltpu.VMEM_SHARED((NW * NPW, 128), jnp.float32),
            pltpu.VMEM(((NPW + 1) * D2,), jnp.float32),
            pltpu.VMEM(((NPW + 1) * D2,), jnp.float32),
            pltpu.VMEM((GB, 128), jnp.float32),
            pltpu.VMEM((GB, 128), jnp.float32),
            pltpu.VMEM((CS,), jnp.int32),
            pltpu.VMEM((CS,), jnp.int32),
            pltpu.VMEM((CS,), jnp.float32),
            pltpu.VMEM((16,), jnp.int32),
            pltpu.SemaphoreType.DMA,
            pltpu.SemaphoreType.DMA,
        ],
    )


def _make_linear_body(nagg):
    def body(*refs):
        x_ref = refs[0]
        a_refs = refs[1:1 + nagg]
        w_ref, b_ref, o_ref = refs[1 + nagg:]
        a = jnp.concatenate([r[...] for r in a_refs], axis=1)
        a = jnp.where(jnp.isneginf(a), 0.0, a)
        x = x_ref[...] + a
        y = jnp.dot(x, w_ref[...], preferred_element_type=jnp.float32)
        o_ref[...] = jnp.maximum(y + b_ref[...], 0.0)
    return body


def _linear(x, aggs, W, b):
    n, din = x.shape
    dk = din // len(aggs)
    dout = W.shape[1]
    bn = 1000
    aspec = pl.BlockSpec((bn, dk), lambda i: (i, 0))
    return pl.pallas_call(
        _make_linear_body(len(aggs)),
        grid=(n // bn,),
        in_specs=[pl.BlockSpec((bn, din), lambda i: (i, 0))]
        + [aspec] * len(aggs)
        + [
            pl.BlockSpec((din, dout), lambda i: (0, 0)),
            pl.BlockSpec((1, dout), lambda i: (0, 0)),
        ],
        out_specs=pl.BlockSpec((bn, dout), lambda i: (i, 0)),
        out_shape=jax.ShapeDtypeStruct((n, dout), jnp.float32),
    )(x, *aggs, W, b.reshape(1, dout))


def kernel(feat, edge_index, edge_weight, W1, b1, W2, b2, W3, b3, W4, b4):
    n = feat.shape[0]
    src = edge_index[0]
    dst = edge_index[1]
    bsrc, bdst, bew, cnt = _make_bucket(src.shape[0])(src, dst, edge_weight)
    h = feat
    for W, b in ((W1, b1), (W2, b2), (W3, b3), (W4, b4)):
        din = W.shape[0]
        hg = jnp.pad(h, ((0, NW * NPW - n), (0, 128 - din if din < 128 else 0)))
        aggs = _make_agg(din)(hg, bsrc, bdst, bew, cnt)
        dk = min(din, 128) // 2
        aggs = [a.reshape(NW * NPW, dk)[:n] for a in aggs]
        h = _linear(h, aggs, W, b)
    return h


# speedup vs baseline: 1.1869x; 1.0283x over previous
import functools

import jax
import jax.numpy as jnp
from jax import lax
from jax.experimental import pallas as pl
from jax.experimental.pallas import tpu as pltpu
from jax.experimental.pallas import tpu_sc as plsc

NC = 2
NS = 16
NW = NC * NS
NPW = 320
DUMMY = NPW
C0 = 20000
CS = 1024
GB = 16
ECAP = 2048 * 80

NEG_INF = float("-inf")


def _worker_id():
    return lax.axis_index("s") * NC + lax.axis_index("c")


def _bucket_body(src_hbm, dst_hbm, ew_hbm, bsrc_hbm, bdst_hbm, bew_hbm,
                 cnt_hbm, srcc, dstc, ewc, csrc, cdst, cew, cntv):
    w = _worker_id()
    lo = w * NPW
    lanes = lax.iota(jnp.int32, 16)
    nchunks = src_hbm.shape[0] // C0

    def chunk_body(chunk, total):
        base = pl.multiple_of(chunk * C0, 8)
        pltpu.sync_copy(src_hbm.at[pl.ds(base, C0)], srcc)
        pltpu.sync_copy(dst_hbm.at[pl.ds(base, C0)], dstc)
        pltpu.sync_copy(ew_hbm.at[pl.ds(base, C0)], ewc)

        def scan_body(i, m):
            off = i * 32
            dv1 = dstc[pl.ds(off, 16)]
            dv2 = dstc[pl.ds(off + 16, 16)]
            msk1 = (dv1 >= lo) & (dv1 < lo + NPW)
            msk2 = (dv2 >= lo) & (dv2 < lo + NPW)
            c1 = plsc.cumsum(jnp.where(msk1, 1, 0))
            c2 = plsc.cumsum(jnp.where(msk2, 1, 0))
            t1 = c1[15]
            pos1 = m + c1 - 1
            pos2 = m + t1 + c2 - 1
            plsc.store_scatter(cdst, [pos1], dv1 - lo, mask=msk1)
            plsc.store_scatter(cdst, [pos2], dv2 - lo, mask=msk2)
            sv1 = srcc[pl.ds(off, 16)]
            sv2 = srcc[pl.ds(off + 16, 16)]
            plsc.store_scatter(csrc, [pos1], sv1, mask=msk1)
            plsc.store_scatter(csrc, [pos2], sv2, mask=msk2)
            ev1 = ewc[pl.ds(off, 16)]
            ev2 = ewc[pl.ds(off + 16, 16)]
            plsc.store_scatter(cew, [pos1], ev1, mask=msk1)
            plsc.store_scatter(cew, [pos2], ev2, mask=msk2)
            return m + t1 + c2[15]

        m = lax.fori_loop(0, C0 // 32, scan_body, jnp.int32(0))
        cdst[pl.ds(m, 16)] = jnp.full((16,), DUMMY, jnp.int32)
        cdst[pl.ds(m + 16, 16)] = jnp.full((16,), DUMMY, jnp.int32)
        csrc[pl.ds(m, 16)] = jnp.zeros((16,), jnp.int32)
        csrc[pl.ds(m + 16, 16)] = jnp.zeros((16,), jnp.int32)
        cew[pl.ds(m, 16)] = jnp.zeros((16,), jnp.float32)
        cew[pl.ds(m + 16, 16)] = jnp.zeros((16,), jnp.float32)
        m_p = ((m + GB - 1) // GB) * GB

        def flush_big(k, _):
            o = k * 256
            fo = pl.multiple_of(w * ECAP + total + o, 8)
            pltpu.sync_copy(csrc.at[pl.ds(o, 256)],
                            bsrc_hbm.at[pl.ds(fo, 256)])
            pltpu.sync_copy(cdst.at[pl.ds(o, 256)],
                            bdst_hbm.at[pl.ds(fo, 256)])
            pltpu.sync_copy(cew.at[pl.ds(o, 256)], bew_hbm.at[pl.ds(fo, 256)])
            return 0

        nbig = m_p // 256
        lax.fori_loop(0, nbig, flush_big, 0)

        def flush_body(k, _):
            o = nbig * 256 + k * 16
            fo = pl.multiple_of(w * ECAP + total + o, 8)
            pltpu.sync_copy(csrc.at[pl.ds(o, 16)], bsrc_hbm.at[pl.ds(fo, 16)])
            pltpu.sync_copy(cdst.at[pl.ds(o, 16)], bdst_hbm.at[pl.ds(fo, 16)])
            pltpu.sync_copy(cew.at[pl.ds(o, 16)], bew_hbm.at[pl.ds(fo, 16)])
            return 0

        lax.fori_loop(0, (m_p - nbig * 256) // 16, flush_body, 0)
        return total + m_p

    total = lax.fori_loop(0, nchunks, chunk_body, jnp.int32(0))
    cntv[...] = jnp.where(lanes == 0, total, 0)
    pltpu.sync_copy(cntv, cnt_hbm.at[pl.ds(pl.multiple_of(w * 16, 8), 16)])


def _make_bucket(E):
    mesh = plsc.VectorSubcoreMesh(core_axis_name="c", subcore_axis_name="s")
    return pl.kernel(
        _bucket_body,
        out_type=(
            jax.ShapeDtypeStruct((NW * ECAP,), jnp.int32),
            jax.ShapeDtypeStruct((NW * ECAP,), jnp.int32),
            jax.ShapeDtypeStruct((NW * ECAP,), jnp.float32),
            jax.ShapeDtypeStruct((NW * 16,), jnp.int32),
        ),
        mesh=mesh,
        compiler_params=pltpu.CompilerParams(needs_layout_passes=False),
        scratch_types=[
            pltpu.VMEM((C0,), jnp.int32),
            pltpu.VMEM((C0,), jnp.int32),
            pltpu.VMEM((C0,), jnp.float32),
            pltpu.VMEM((C0 + GB,), jnp.int32),
            pltpu.VMEM((C0 + GB,), jnp.int32),
            pltpu.VMEM((C0 + GB,), jnp.float32),
            pltpu.VMEM((16,), jnp.int32),
        ],
    )


def _agg_body(D, feat_hbm, bsrc_hbm, bdst_hbm, bew_hbm, cnt_hbm, *rest):
    nout = 4 if D > 128 else 2
    agg_hbms = rest[:nout]
    (shared, agg0, agg1, rows_a, rows_b,
     srcs, dsts, ews, cntv, sem_a, sem_b) = rest[nout:]
    w = _worker_id()
    lo = w * NPW
    s_id = lax.axis_index("s")
    DP = min(D, 128)
    D2 = DP // 2
    npass = max(1, D // 128)
    aggs = (agg0, agg1)

    pltpu.sync_copy(cnt_hbm.at[pl.ds(pl.multiple_of(w * 16, 8), 16)], cntv)
    total = cntv[...][0]

    def fire(g, rows, sem):
        sv = srcs[pl.ds(g * GB, GB)]
        return pltpu.async_copy(shared.at[sv], rows, sem)

    def process(g, rows):
        off = g * GB
        for q in range(GB // 16):
            av = dsts[pl.ds(off + q * 16, 16)] * D2
            ev = ews[pl.ds(off + q * 16, 16)]
            for e in range(16):
                e_s = ev[e]
                abase = av[e]
                for j in range(D2 // 16):
                    for k in range(2):
                        a = aggs[k][pl.ds(abase + j * 16, 16)]
                        r = rows[q * 16 + e, pl.ds(k * D2 + j * 16, 16)]
                        aggs[k][pl.ds(abase + j * 16, 16)] = jnp.maximum(
                            a, r * e_s)

    def chunk_body(c, _):
        cbase = c * CS
        co = pl.multiple_of(w * ECAP + cbase, 8)
        pltpu.sync_copy(bsrc_hbm.at[pl.ds(co, CS)], srcs)
        pltpu.sync_copy(bdst_hbm.at[pl.ds(co, CS)], dsts)
        pltpu.sync_copy(bew_hbm.at[pl.ds(co, CS)], ews)
        nb = jnp.minimum(CS, total - cbase) // GB

        @pl.when(nb > 0)
        def _():
            fire(0, rows_a, sem_a)

        def pair_body(p, _):
            ga = 2 * p
            gb = 2 * p + 1

            @pl.when(gb < nb)
            def _():
                fire(gb, rows_b, sem_b)

            pltpu.make_async_copy(shared.at[srcs[pl.ds(0, GB)]],
                                  rows_a, sem_a).wait()
            process(ga, rows_a)

            @pl.when(ga + 2 < nb)
            def _():
                fire(ga + 2, rows_a, sem_a)

            @pl.when(gb < nb)
            def _():
                pltpu.make_async_copy(shared.at[srcs[pl.ds(0, GB)]],
                                      rows_b, sem_b).wait()
                process(gb, rows_b)

            return 0

        lax.fori_loop(0, (nb + 1) // 2, pair_body, 0)
        return 0

    ro = pl.multiple_of(s_id * (NW * NPW // NS), 8)
    nrs = NW * NPW // NS
    for p in range(npass):
        if p > 0:
            plsc.subcore_barrier()
        if D > 128:
            pltpu.sync_copy(feat_hbm.at[pl.ds(ro, nrs), pl.ds(128 * p, 128)],
                            shared.at[pl.ds(ro, nrs)])
        else:
            pltpu.sync_copy(feat_hbm.at[pl.ds(ro, nrs)],
                            shared.at[pl.ds(ro, nrs)])
        plsc.subcore_barrier()

        def init_body(i, _):
            ninf = jnp.full((16,), NEG_INF, jnp.float32)
            for k in range(2):
                aggs[k][pl.ds(i * 16, 16)] = ninf
            return 0

        lax.fori_loop(0, (NPW + 1) * D2 // 16, init_body, 0)

        nchunks = (total + CS - 1) // CS
        lax.fori_loop(0, nchunks, chunk_body, 0)

        for k in range(2):
            pltpu.sync_copy(
                aggs[k].at[pl.ds(0, NPW * D2)],
                agg_hbms[2 * p + k].at[
                    pl.ds(pl.multiple_of(lo * D2, 8), NPW * D2)])


def _make_agg(D):
    nout = 4 if D > 128 else 2
    D2 = min(D, 128) // 2
    mesh = plsc.VectorSubcoreMesh(core_axis_name="c", subcore_axis_name="s")
    return pl.kernel(
        functools.partial(_agg_body, D),
        out_type=tuple(
            jax.ShapeDtypeStruct((NW * NPW * D2,), jnp.float32)
            for _ in range(nout)),
        mesh=mesh,
        compiler_params=pltpu.CompilerParams(needs_layout_passes=False),
        scratch_types=[
            pltpu.VMEM_SHARED((NW * NPW, 128), jnp.float32),
            pltpu.VMEM(((NPW + 1) * D2,), jnp.float32),
            pltpu.VMEM(((NPW + 1) * D2,), jnp.float32),
            pltpu.VMEM((GB, 128), jnp.float32),
            pltpu.VMEM((GB, 128), jnp.float32),
            pltpu.VMEM((CS,), jnp.int32),
            pltpu.VMEM((CS,), jnp.int32),
            pltpu.VMEM((CS,), jnp.float32),
            pltpu.VMEM((16,), jnp.int32),
            pltpu.SemaphoreType.DMA,
            pltpu.SemaphoreType.DMA,
        ],
    )


def _make_linear_body(nagg):
    def body(*refs):
        x_ref = refs[0]
        a_refs = refs[1:1 + nagg]
        w_ref, b_ref, o_ref = refs[1 + nagg:]
        a = jnp.concatenate([r[...] for r in a_refs], axis=1)
        a = jnp.where(jnp.isneginf(a), 0.0, a)
        x = x_ref[...] + a
        y = jnp.dot(x, w_ref[...], preferred_element_type=jnp.float32)
        o_ref[...] = jnp.maximum(y + b_ref[...], 0.0)
    return body


def _linear(x, aggs, W, b):
    n, din = x.shape
    dk = din // len(aggs)
    dout = W.shape[1]
    bn = 1000
    aspec = pl.BlockSpec((bn, dk), lambda i: (i, 0))
    return pl.pallas_call(
        _make_linear_body(len(aggs)),
        grid=(n // bn,),
        in_specs=[pl.BlockSpec((bn, din), lambda i: (i, 0))]
        + [aspec] * len(aggs)
        + [
            pl.BlockSpec((din, dout), lambda i: (0, 0)),
            pl.BlockSpec((1, dout), lambda i: (0, 0)),
        ],
        out_specs=pl.BlockSpec((bn, dout), lambda i: (i, 0)),
        out_shape=jax.ShapeDtypeStruct((n, dout), jnp.float32),
    )(x, *aggs, W, b.reshape(1, dout))


def kernel(feat, edge_index, edge_weight, W1, b1, W2, b2, W3, b3, W4, b4):
    n = feat.shape[0]
    src = edge_index[0]
    dst = edge_index[1]
    bsrc, bdst, bew, cnt = _make_bucket(src.shape[0])(src, dst, edge_weight)
    h = feat
    for W, b in ((W1, b1), (W2, b2), (W3, b3), (W4, b4)):
        din = W.shape[0]
        hg = jnp.pad(h, ((0, NW * NPW - n), (0, 128 - din if din < 128 else 0)))
        aggs = _make_agg(din)(hg, bsrc, bdst, bew, cnt)
        dk = min(din, 128) // 2
        aggs = [a.reshape(NW * NPW, dk)[:n] for a in aggs]
        h = _linear(h, aggs, W, b)
    return h
